# baseline (speedup 1.0000x reference)
import functools

import jax
import jax.numpy as jnp
from jax import lax
from jax.experimental import pallas as pl
from jax.experimental.pallas import tpu as pltpu
from jax.experimental.pallas import tpu_sc as plsc

N = 10000
E = 320000
N_PAD = 10240
E_PAD = 344064
NC = 2
NS = 16
NW = NC * NS
CHUNK = 128
CPW = E_PAD // (NW * CHUNK)
RPT = N_PAD // NS
NBP = 12
GDP = 6
NBF = 6
GDF = 3

_mesh = plsc.VectorSubcoreMesh(
    core_axis_name="c", subcore_axis_name="s", num_cores=NC, num_subcores=NS)
_sc_params = pltpu.CompilerParams(use_tc_tiling_on_sc=False)
_sc_params_nolayout = pltpu.CompilerParams(use_tc_tiling_on_sc=False,
                                           needs_layout_passes=False)


@functools.partial(
    pl.kernel,
    out_type=jax.ShapeDtypeStruct((NW, N_PAD), jnp.float32),
    mesh=_mesh,
    scratch_types=[
        pltpu.VMEM((CPW, CHUNK), jnp.int32),
        pltpu.VMEM((N_PAD,), jnp.float32),
        pltpu.SemaphoreType.DMA,
    ],
    compiler_params=_sc_params_nolayout,
)
def _deg_kernel(ei_hbm, out_hbm, idx_v, hist_v, sem):
    c = lax.axis_index("c")
    s = lax.axis_index("s")
    wid = c * NS + s
    pltpu.async_copy(ei_hbm.at[1, pl.ds(wid * CPW, CPW)], idx_v, sem)

    zeros_vec = jnp.zeros((16,), jnp.float32)

    def zbody(i, carry):
        hist_v[pl.ds(i * 16, 16)] = zeros_vec
        return carry

    lax.fori_loop(0, N_PAD // 16, zbody, 0)
    pltpu.make_async_copy(ei_hbm.at[1, pl.ds(wid * CPW, CPW)], idx_v, sem).wait()

    ones_vec = jnp.ones((16,), jnp.float32)

    def body(j, carry):
        for k in range(CHUNK // 16):
            v = idx_v[j, pl.ds(16 * k, 16)]
            plsc.addupdate_scatter(hist_v, [v], ones_vec)
        return carry

    lax.fori_loop(0, CPW, body, 0)
    pltpu.sync_copy(hist_v, out_hbm.at[wid])


def _edge_ring(table_sh, acc_sh, src_v, dst_v, rows, gsem, ssem, nb, gd):
    def gather(jt, b):
        pltpu.async_copy(table_sh.at[src_v.at[jt]], rows[b], gsem[b])

    def gwait(j, b):
        pltpu.make_async_copy(table_sh.at[src_v.at[j]], rows[b], gsem[b]).wait()

    def scatter(j, b):
        pltpu.async_copy(rows[b], acc_sh.at[dst_v.at[j]], ssem[b], add=True)

    def swait(j, b):
        pltpu.make_async_copy(rows[b], acc_sh.at[dst_v.at[j]], ssem[b]).wait()

    for b in range(gd):
        gather(b, b)

    for j in range(nb):
        b = j % nb
        gwait(j, b)
        scatter(j, b)
        jt = j + gd
        bt = jt % nb
        if jt >= nb:
            swait(jt - nb, bt)
        gather(jt, bt)

    def outer(g, carry):
        base = g * nb
        for b in range(nb):
            j = base + b
            gwait(j, b)
            scatter(j, b)
            jt = j + gd
            bt = (b + gd) % nb
            swait(jt - nb, bt)
            gather(jt, bt)
        return carry

    lax.fori_loop(1, CPW // nb - 1, outer, 0)

    for b in range(nb):
        j = CPW - nb + b
        gwait(j, b)
        scatter(j, b)
        jt = j + gd
        if jt < CPW:
            bt = jt % nb
            swait(jt - nb, bt)
            gather(jt, bt)

    for b in range(nb):
        swait(CPW - nb + b, b)


def _prop_scratch(nb):
    return [
        pltpu.VMEM((CPW, CHUNK), jnp.int32),
        pltpu.VMEM((CPW, CHUNK), jnp.int32),
        [pltpu.VMEM((CHUNK, 32), jnp.float32)] * nb,
        [pltpu.SemaphoreType.DMA] * nb,
        [pltpu.SemaphoreType.DMA] * nb,
        pltpu.VMEM_SHARED((N_PAD, 32), jnp.float32),
        pltpu.VMEM_SHARED((N_PAD, 32), jnp.float32),
    ]


@functools.partial(
    pl.kernel,
    out_type=jax.ShapeDtypeStruct((NC, N_PAD, 32), jnp.float32),
    mesh=_mesh,
    scratch_types=_prop_scratch(NBP),
    compiler_params=_sc_params,
)
def _prop_kernel(table_hbm, ei_hbm, zeros32_hbm, out_hbm,
                 src_v, dst_v, rows, gsem, ssem, acc_sh, table_sh):
    c = lax.axis_index("c")
    s = lax.axis_index("s")
    wid = c * NS + s
    rbase = s * RPT
    pltpu.async_copy(zeros32_hbm.at[pl.ds(rbase, RPT)], acc_sh.at[pl.ds(rbase, RPT)], gsem[0])
    pltpu.async_copy(table_hbm.at[pl.ds(rbase, RPT)], table_sh.at[pl.ds(rbase, RPT)], gsem[1])
    pltpu.async_copy(ei_hbm.at[0, pl.ds(wid * CPW, CPW)], src_v, gsem[2])
    pltpu.async_copy(ei_hbm.at[1, pl.ds(wid * CPW, CPW)], dst_v, gsem[3])
    pltpu.make_async_copy(zeros32_hbm.at[pl.ds(rbase, RPT)], acc_sh.at[pl.ds(rbase, RPT)], gsem[0]).wait()
    pltpu.make_async_copy(table_hbm.at[pl.ds(rbase, RPT)], table_sh.at[pl.ds(rbase, RPT)], gsem[1]).wait()
    pltpu.make_async_copy(ei_hbm.at[0, pl.ds(wid * CPW, CPW)], src_v, gsem[2]).wait()
    pltpu.make_async_copy(ei_hbm.at[1, pl.ds(wid * CPW, CPW)], dst_v, gsem[3]).wait()
    plsc.subcore_barrier()
    _edge_ring(table_sh, acc_sh, src_v, dst_v, rows, gsem, ssem, NBP, GDP)
    plsc.subcore_barrier()
    pltpu.sync_copy(acc_sh.at[pl.ds(rbase, RPT)], out_hbm.at[c, pl.ds(rbase, RPT)])


def _elu(a):
    return jnp.where(a > 0, a, jnp.exp(jnp.minimum(a, 0.0)) - 1.0)


def _tc_stage1(degp_hbm, x_ref, w1_ref, table_hbm, dinv_ref,
               degp_v, tbl_v, sem, sem2):
    pltpu.async_copy(degp_hbm, degp_v, sem).wait()
    degs = lax.dot_general(degp_v[...], jnp.ones((NW, 1), jnp.float32),
                           (((0,), (0,)), ((), ())),
                           preferred_element_type=jnp.float32)
    dinv = lax.rsqrt(degs[0:N])
    dinv_ref[...] = dinv
    t1 = jnp.dot(x_ref[...], w1_ref[...], preferred_element_type=jnp.float32)
    tbl_v[...] = t1 * dinv
    pltpu.async_copy(tbl_v, table_hbm.at[pl.ds(0, N)], sem2).wait()


def _tc_stage2(pp_hbm, dinv_ref, b1_ref, table_hbm, pp_v, tbl_v, sem, sem2):
    pltpu.async_copy(pp_hbm, pp_v, sem).wait()
    dinv = dinv_ref[...]
    h1 = _elu((pp_v[0, 0:N] + pp_v[1, 0:N]) * dinv + b1_ref[...])
    tbl_v[...] = h1 * dinv
    pltpu.async_copy(tbl_v, table_hbm.at[pl.ds(0, N)], sem2).wait()


def _tc_stage3(pp_hbm, dinv_ref, w2_ref, b2_ref, w4_ref, table_hbm,
               pp_v, tbl_v, sem, sem2):
    pltpu.async_copy(pp_hbm, pp_v, sem).wait()
    dinv = dinv_ref[...]
    p = (pp_v[0, 0:N] + pp_v[1, 0:N]) * dinv
    h2 = _elu(jnp.dot(p, w2_ref[...], preferred_element_type=jnp.float32) + b2_ref[...])
    t3 = jnp.dot(h2, w4_ref[...], preferred_element_type=jnp.float32)
    tbl_v[...] = t3 * dinv
    pltpu.async_copy(tbl_v, table_hbm.at[pl.ds(0, N)], sem2).wait()


def _tc_stage4(pp_hbm, dinv_ref, b4_ref, out_ref, pp_v, sem):
    pltpu.async_copy(pp_hbm, pp_v, sem).wait()
    dinv = dinv_ref[...]
    logits = (pp_v[0, 0:N] + pp_v[1, 0:N]) * dinv + b4_ref[...]
    col = lax.broadcasted_iota(jnp.int32, logits.shape, 1)
    z = jnp.where(col < 19, logits, -jnp.inf)
    zmax = jnp.max(z, axis=1, keepdims=True)
    e = jnp.exp(z - zmax)
    out_ref[...] = (e / jnp.sum(e, axis=1, keepdims=True))[:, 0:19]


def _sds(shape):
    return jax.ShapeDtypeStruct(shape, jnp.float32)


_HBM_SPEC = pl.BlockSpec(memory_space=pltpu.HBM)
_VMEM_SPEC = pl.BlockSpec(memory_space=pltpu.VMEM)


def kernel(x, edge_index, W1, b1, W2, b2, W4, b4):
    loop = jnp.arange(N_PAD, dtype=jnp.int32)
    ei3d = jnp.concatenate(
        [edge_index, jnp.stack([loop, loop]),
         jnp.full((2, E_PAD - E - N_PAD), N_PAD - 1, jnp.int32)],
        axis=1).reshape(2, E_PAD // CHUNK, CHUNK)
    zeros32 = jnp.zeros((N_PAD, 32), jnp.float32)
    W4p = jnp.zeros((64, 32), jnp.float32).at[:, :19].set(W4)
    b1r = b1.reshape(1, 32)
    b2r = b2.reshape(1, 64)
    b4r = jnp.zeros((1, 32), jnp.float32).at[0, :19].set(b4)

    degp = _deg_kernel(ei3d)

    table1, dinv = pl.pallas_call(
        _tc_stage1,
        out_shape=[_sds((N_PAD, 32)), _sds((N, 1))],
        in_specs=[_HBM_SPEC, _VMEM_SPEC, _VMEM_SPEC],
        out_specs=[_HBM_SPEC, _VMEM_SPEC],
        scratch_shapes=[pltpu.VMEM((NW, N_PAD), jnp.float32),
                        pltpu.VMEM((N, 32), jnp.float32),
                        pltpu.SemaphoreType.DMA, pltpu.SemaphoreType.DMA],
    )(degp, x, W1)

    pp1 = _prop_kernel(table1, ei3d, zeros32)

    table2 = pl.pallas_call(
        _tc_stage2,
        out_shape=_sds((N_PAD, 32)),
        in_specs=[_HBM_SPEC, _VMEM_SPEC, _VMEM_SPEC],
        out_specs=_HBM_SPEC,
        scratch_shapes=[pltpu.VMEM((NC, N_PAD, 32), jnp.float32),
                        pltpu.VMEM((N, 32), jnp.float32),
                        pltpu.SemaphoreType.DMA, pltpu.SemaphoreType.DMA],
    )(pp1, dinv, b1r)

    pp2 = _prop_kernel(table2, ei3d, zeros32)

    table3 = pl.pallas_call(
        _tc_stage3,
        out_shape=_sds((N_PAD, 32)),
        in_specs=[_HBM_SPEC] + [_VMEM_SPEC] * 4,
        out_specs=_HBM_SPEC,
        scratch_shapes=[pltpu.VMEM((NC, N_PAD, 32), jnp.float32),
                        pltpu.VMEM((N, 32), jnp.float32),
                        pltpu.SemaphoreType.DMA, pltpu.SemaphoreType.DMA],
    )(pp2, dinv, W2, b2r, W4p)

    pp3 = _prop_kernel(table3, ei3d, zeros32)

    probs = pl.pallas_call(
        _tc_stage4,
        out_shape=_sds((N, 19)),
        in_specs=[_HBM_SPEC, _VMEM_SPEC, _VMEM_SPEC],
        out_specs=_VMEM_SPEC,
        scratch_shapes=[pltpu.VMEM((NC, N_PAD, 32), jnp.float32),
                        pltpu.SemaphoreType.DMA],
    )(pp3, dinv, b4r)

    return probs


# speedup vs baseline: 1.2591x; 1.2591x over previous
import functools

import jax
import jax.numpy as jnp
from jax import lax
from jax.experimental import pallas as pl
from jax.experimental.pallas import tpu as pltpu
from jax.experimental.pallas import tpu_sc as plsc

N = 10000
E = 320000
N_PAD = 10240
E_PAD = 331776
NC = 2
NS = 16
NW = NC * NS
CHUNK = 128
CPW = E_PAD // (NW * CHUNK)
RPT = N_PAD // NS
NBP = 9
GDP = 4

_mesh = plsc.VectorSubcoreMesh(
    core_axis_name="c", subcore_axis_name="s", num_cores=NC, num_subcores=NS)
_sc_params = pltpu.CompilerParams(use_tc_tiling_on_sc=False)
_sc_params_nolayout = pltpu.CompilerParams(use_tc_tiling_on_sc=False,
                                           needs_layout_passes=False)


@functools.partial(
    pl.kernel,
    out_type=jax.ShapeDtypeStruct((NW, N_PAD), jnp.float32),
    mesh=_mesh,
    scratch_types=[
        pltpu.VMEM((CPW, CHUNK), jnp.int32),
        pltpu.VMEM((N_PAD,), jnp.float32),
        pltpu.SemaphoreType.DMA,
    ],
    compiler_params=_sc_params_nolayout,
)
def _deg_kernel(ei_hbm, out_hbm, idx_v, hist_v, sem):
    c = lax.axis_index("c")
    s = lax.axis_index("s")
    wid = c * NS + s
    pltpu.async_copy(ei_hbm.at[1, pl.ds(wid * CPW, CPW)], idx_v, sem)

    zeros_vec = jnp.zeros((16,), jnp.float32)

    def zbody(i, carry):
        hist_v[pl.ds(i * 16, 16)] = zeros_vec
        return carry

    lax.fori_loop(0, N_PAD // 16, zbody, 0)
    pltpu.make_async_copy(ei_hbm.at[1, pl.ds(wid * CPW, CPW)], idx_v, sem).wait()

    ones_vec = jnp.ones((16,), jnp.float32)

    def body(j, carry):
        for k in range(CHUNK // 16):
            v = idx_v[j, pl.ds(16 * k, 16)]
            plsc.addupdate_scatter(hist_v, [v], ones_vec)
        return carry

    lax.fori_loop(0, CPW, body, 0)
    pltpu.sync_copy(hist_v, out_hbm.at[wid])


def _edge_ring(table_sh, acc_sh, src_v, dst_v, rows, gsem, ssem, nb, gd):
    def gather(jt, b):
        pltpu.async_copy(table_sh.at[src_v.at[jt]], rows[b], gsem[b])

    def gwait(j, b):
        pltpu.make_async_copy(table_sh.at[src_v.at[j]], rows[b], gsem[b]).wait()

    def scatter(j, b):
        pltpu.async_copy(rows[b], acc_sh.at[dst_v.at[j]], ssem[b], add=True)

    def swait(j, b):
        pltpu.make_async_copy(rows[b], acc_sh.at[dst_v.at[j]], ssem[b]).wait()

    for b in range(gd):
        gather(b, b)

    for j in range(nb):
        b = j % nb
        gwait(j, b)
        scatter(j, b)
        jt = j + gd
        bt = jt % nb
        if jt >= nb:
            swait(jt - nb, bt)
        gather(jt, bt)

    def outer(g, carry):
        base = g * nb
        for b in range(nb):
            j = base + b
            gwait(j, b)
            scatter(j, b)
            jt = j + gd
            bt = (b + gd) % nb
            swait(jt - nb, bt)
            gather(jt, bt)
        return carry

    lax.fori_loop(1, CPW // nb - 1, outer, 0)

    for b in range(nb):
        j = CPW - nb + b
        gwait(j, b)
        scatter(j, b)
        jt = j + gd
        if jt < CPW:
            bt = jt % nb
            swait(jt - nb, bt)
            gather(jt, bt)

    for b in range(nb):
        swait(CPW - nb + b, b)


def _prop_scratch(nb):
    return [
        pltpu.VMEM((CPW, CHUNK), jnp.int32),
        pltpu.VMEM((CPW, CHUNK), jnp.int32),
        [pltpu.VMEM((CHUNK, 32), jnp.float32)] * nb,
        [pltpu.SemaphoreType.DMA] * nb,
        [pltpu.SemaphoreType.DMA] * nb,
        pltpu.VMEM_SHARED((N_PAD, 32), jnp.float32),
        pltpu.VMEM_SHARED((N_PAD, 32), jnp.float32),
    ]


@functools.partial(
    pl.kernel,
    out_type=jax.ShapeDtypeStruct((NC, N_PAD, 32), jnp.float32),
    mesh=_mesh,
    scratch_types=_prop_scratch(NBP),
    compiler_params=_sc_params,
)
def _prop_kernel(table_hbm, ei_hbm, zeros32_hbm, out_hbm,
                 src_v, dst_v, rows, gsem, ssem, acc_sh, table_sh):
    c = lax.axis_index("c")
    s = lax.axis_index("s")
    wid = c * NS + s
    rbase = s * RPT
    pltpu.async_copy(zeros32_hbm.at[pl.ds(rbase, RPT)], acc_sh.at[pl.ds(rbase, RPT)], gsem[0])
    pltpu.async_copy(table_hbm.at[pl.ds(rbase, RPT)], table_sh.at[pl.ds(rbase, RPT)], gsem[1])
    pltpu.async_copy(ei_hbm.at[0, pl.ds(wid * CPW, CPW)], src_v, gsem[2])
    pltpu.async_copy(ei_hbm.at[1, pl.ds(wid * CPW, CPW)], dst_v, gsem[3])
    pltpu.make_async_copy(zeros32_hbm.at[pl.ds(rbase, RPT)], acc_sh.at[pl.ds(rbase, RPT)], gsem[0]).wait()
    pltpu.make_async_copy(table_hbm.at[pl.ds(rbase, RPT)], table_sh.at[pl.ds(rbase, RPT)], gsem[1]).wait()
    pltpu.make_async_copy(ei_hbm.at[0, pl.ds(wid * CPW, CPW)], src_v, gsem[2]).wait()
    pltpu.make_async_copy(ei_hbm.at[1, pl.ds(wid * CPW, CPW)], dst_v, gsem[3]).wait()
    plsc.subcore_barrier()
    _edge_ring(table_sh, acc_sh, src_v, dst_v, rows, gsem, ssem, NBP, GDP)
    plsc.subcore_barrier()
    pltpu.sync_copy(acc_sh.at[pl.ds(rbase, RPT)], out_hbm.at[c, pl.ds(rbase, RPT)])


def _elu(a):
    return jnp.where(a > 0, a, jnp.exp(jnp.minimum(a, 0.0)) - 1.0)


def _tc_stage1(degp_hbm, x_ref, w1_ref, table_hbm, dinv_ref,
               degp_v, tbl_v, sem, sem2):
    pltpu.async_copy(degp_hbm, degp_v, sem).wait()
    degs = lax.dot_general(degp_v[...], jnp.ones((NW, 1), jnp.float32),
                           (((0,), (0,)), ((), ())),
                           preferred_element_type=jnp.float32)
    dinv = lax.rsqrt(degs[0:N])
    dinv_ref[...] = dinv
    t1 = jnp.dot(x_ref[...], w1_ref[...], preferred_element_type=jnp.float32)
    tbl_v[...] = t1 * dinv
    pltpu.async_copy(tbl_v, table_hbm.at[pl.ds(0, N)], sem2).wait()


def _tc_stage2(pp_hbm, dinv_ref, b1_ref, table_hbm, pp_v, tbl_v, sem, sem2):
    pltpu.async_copy(pp_hbm, pp_v, sem).wait()
    dinv = dinv_ref[...]
    h1 = _elu((pp_v[0, 0:N] + pp_v[1, 0:N]) * dinv + b1_ref[...])
    tbl_v[...] = h1 * dinv
    pltpu.async_copy(tbl_v, table_hbm.at[pl.ds(0, N)], sem2).wait()


def _tc_stage3(pp_hbm, dinv_ref, w2_ref, b2_ref, w4_ref, table_hbm,
               pp_v, tbl_v, sem, sem2):
    pltpu.async_copy(pp_hbm, pp_v, sem).wait()
    dinv = dinv_ref[...]
    p = (pp_v[0, 0:N] + pp_v[1, 0:N]) * dinv
    h2 = _elu(jnp.dot(p, w2_ref[...], preferred_element_type=jnp.float32) + b2_ref[...])
    t3 = jnp.dot(h2, w4_ref[...], preferred_element_type=jnp.float32)
    tbl_v[...] = t3 * dinv
    pltpu.async_copy(tbl_v, table_hbm.at[pl.ds(0, N)], sem2).wait()


def _tc_stage4(pp_hbm, dinv_ref, b4_ref, out_ref, pp_v, sem):
    pltpu.async_copy(pp_hbm, pp_v, sem).wait()
    dinv = dinv_ref[...]
    logits = (pp_v[0, 0:N] + pp_v[1, 0:N]) * dinv + b4_ref[...]
    col = lax.broadcasted_iota(jnp.int32, logits.shape, 1)
    z = jnp.where(col < 19, logits, -jnp.inf)
    zmax = jnp.max(z, axis=1, keepdims=True)
    e = jnp.exp(z - zmax)
    out_ref[...] = (e / jnp.sum(e, axis=1, keepdims=True))[:, 0:19]


def _sds(shape):
    return jax.ShapeDtypeStruct(shape, jnp.float32)


_HBM_SPEC = pl.BlockSpec(memory_space=pltpu.HBM)
_VMEM_SPEC = pl.BlockSpec(memory_space=pltpu.VMEM)


def kernel(x, edge_index, W1, b1, W2, b2, W4, b4):
    loop = jnp.arange(N_PAD, dtype=jnp.int32)
    dummy = N + jnp.arange(E_PAD - E - N_PAD, dtype=jnp.int32) % (N_PAD - N)
    ei3d = jnp.concatenate(
        [edge_index, jnp.stack([loop, loop]), jnp.stack([dummy, dummy])],
        axis=1).reshape(2, E_PAD // CHUNK, CHUNK)
    zeros32 = jnp.zeros((N_PAD, 32), jnp.float32)
    W4p = jnp.zeros((64, 32), jnp.float32).at[:, :19].set(W4)
    b1r = b1.reshape(1, 32)
    b2r = b2.reshape(1, 64)
    b4r = jnp.zeros((1, 32), jnp.float32).at[0, :19].set(b4)

    degp = _deg_kernel(ei3d)

    table1, dinv = pl.pallas_call(
        _tc_stage1,
        out_shape=[_sds((N_PAD, 32)), _sds((N, 1))],
        in_specs=[_HBM_SPEC, _VMEM_SPEC, _VMEM_SPEC],
        out_specs=[_HBM_SPEC, _VMEM_SPEC],
        scratch_shapes=[pltpu.VMEM((NW, N_PAD), jnp.float32),
                        pltpu.VMEM((N, 32), jnp.float32),
                        pltpu.SemaphoreType.DMA, pltpu.SemaphoreType.DMA],
    )(degp, x, W1)

    pp1 = _prop_kernel(table1, ei3d, zeros32)

    table2 = pl.pallas_call(
        _tc_stage2,
        out_shape=_sds((N_PAD, 32)),
        in_specs=[_HBM_SPEC, _VMEM_SPEC, _VMEM_SPEC],
        out_specs=_HBM_SPEC,
        scratch_shapes=[pltpu.VMEM((NC, N_PAD, 32), jnp.float32),
                        pltpu.VMEM((N, 32), jnp.float32),
                        pltpu.SemaphoreType.DMA, pltpu.SemaphoreType.DMA],
    )(pp1, dinv, b1r)

    pp2 = _prop_kernel(table2, ei3d, zeros32)

    table3 = pl.pallas_call(
        _tc_stage3,
        out_shape=_sds((N_PAD, 32)),
        in_specs=[_HBM_SPEC] + [_VMEM_SPEC] * 4,
        out_specs=_HBM_SPEC,
        scratch_shapes=[pltpu.VMEM((NC, N_PAD, 32), jnp.float32),
                        pltpu.VMEM((N, 32), jnp.float32),
                        pltpu.SemaphoreType.DMA, pltpu.SemaphoreType.DMA],
    )(pp2, dinv, W2, b2r, W4p)

    pp3 = _prop_kernel(table3, ei3d, zeros32)

    probs = pl.pallas_call(
        _tc_stage4,
        out_shape=_sds((N, 19)),
        in_specs=[_HBM_SPEC, _VMEM_SPEC, _VMEM_SPEC],
        out_specs=_VMEM_SPEC,
        scratch_shapes=[pltpu.VMEM((NC, N_PAD, 32), jnp.float32),
                        pltpu.SemaphoreType.DMA],
    )(pp3, dinv, b4r)

    return probs


# speedup vs baseline: 1.4982x; 1.1899x over previous
import functools

import jax
import jax.numpy as jnp
from jax import lax
from jax.experimental import pallas as pl
from jax.experimental.pallas import tpu as pltpu
from jax.experimental.pallas import tpu_sc as plsc

N = 10000
E = 320000
N_PAD = 10240
E_PAD = 331776
NC = 2
NS = 16
NW = NC * NS
CHUNK = 128
CPW = E_PAD // (NW * CHUNK)
RPT = N_PAD // NS
NBP = 9
GDP = 4

_mesh = plsc.VectorSubcoreMesh(
    core_axis_name="c", subcore_axis_name="s", num_cores=NC, num_subcores=NS)
_sc_params = pltpu.CompilerParams(use_tc_tiling_on_sc=False)
_sc_params_nolayout = pltpu.CompilerParams(use_tc_tiling_on_sc=False,
                                           needs_layout_passes=False)


@functools.partial(
    pl.kernel,
    out_type=jax.ShapeDtypeStruct((NW, N_PAD), jnp.float32),
    mesh=_mesh,
    scratch_types=[
        pltpu.VMEM((CPW, CHUNK), jnp.int32),
        pltpu.VMEM((N_PAD,), jnp.float32),
        pltpu.SemaphoreType.DMA,
    ],
    compiler_params=_sc_params_nolayout,
)
def _deg_kernel(ei_hbm, out_hbm, idx_v, hist_v, sem):
    c = lax.axis_index("c")
    s = lax.axis_index("s")
    wid = c * NS + s
    pltpu.async_copy(ei_hbm.at[1, pl.ds(wid * CPW, CPW)], idx_v, sem)

    zeros_vec = jnp.zeros((16,), jnp.float32)

    def zbody(i, carry):
        hist_v[pl.ds(i * 16, 16)] = zeros_vec
        return carry

    lax.fori_loop(0, N_PAD // 16, zbody, 0)
    pltpu.make_async_copy(ei_hbm.at[1, pl.ds(wid * CPW, CPW)], idx_v, sem).wait()

    ones_vec = jnp.ones((16,), jnp.float32)

    def body(j, carry):
        for k in range(CHUNK // 16):
            v = idx_v[j, pl.ds(16 * k, 16)]
            plsc.addupdate_scatter(hist_v, [v], ones_vec)
        return carry

    lax.fori_loop(0, CPW, body, 0)
    pltpu.sync_copy(hist_v, out_hbm.at[wid])


def _edge_ring(table_sh, acc_sh, src_v, dst_v, rows, gsem, ssem, nb, gd):
    def gather(jt, b):
        pltpu.async_copy(table_sh.at[src_v.at[jt]], rows[b], gsem[b])

    def gwait(j, b):
        pltpu.make_async_copy(table_sh.at[src_v.at[j]], rows[b], gsem[b]).wait()

    def scatter(j, b):
        pltpu.async_copy(rows[b], acc_sh.at[dst_v.at[j]], ssem[b], add=True)

    def swait(j, b):
        pltpu.make_async_copy(rows[b], acc_sh.at[dst_v.at[j]], ssem[b]).wait()

    for b in range(gd):
        gather(b, b)

    for j in range(nb):
        b = j % nb
        gwait(j, b)
        scatter(j, b)
        jt = j + gd
        bt = jt % nb
        if jt >= nb:
            swait(jt - nb, bt)
        gather(jt, bt)

    def outer(g, carry):
        base = g * nb
        for b in range(nb):
            j = base + b
            gwait(j, b)
            scatter(j, b)
            jt = j + gd
            bt = (b + gd) % nb
            swait(jt - nb, bt)
            gather(jt, bt)
        return carry

    lax.fori_loop(1, CPW // nb - 1, outer, 0)

    for b in range(nb):
        j = CPW - nb + b
        gwait(j, b)
        scatter(j, b)
        jt = j + gd
        if jt < CPW:
            bt = jt % nb
            swait(jt - nb, bt)
            gather(jt, bt)

    for b in range(nb):
        swait(CPW - nb + b, b)


def _prop_scratch(nb):
    return [
        pltpu.VMEM((CPW, CHUNK), jnp.int32),
        pltpu.VMEM((CPW, CHUNK), jnp.int32),
        [pltpu.VMEM((CHUNK, 32), jnp.float32)] * nb,
        [pltpu.SemaphoreType.DMA] * nb,
        [pltpu.SemaphoreType.DMA] * nb,
        pltpu.VMEM_SHARED((N_PAD, 32), jnp.float32),
        pltpu.VMEM_SHARED((N_PAD, 32), jnp.float32),
    ]


@functools.partial(
    pl.kernel,
    out_type=jax.ShapeDtypeStruct((NC, N_PAD, 32), jnp.float32),
    mesh=_mesh,
    scratch_types=_prop_scratch(NBP),
    compiler_params=_sc_params,
)
def _prop_kernel(table_hbm, ei_hbm, zeros32_hbm, out_hbm,
                 src_v, dst_v, rows, gsem, ssem, acc_sh, table_sh):
    c = lax.axis_index("c")
    s = lax.axis_index("s")
    wid = c * NS + s
    rbase = s * RPT
    pltpu.async_copy(zeros32_hbm.at[pl.ds(rbase, RPT)], acc_sh.at[pl.ds(rbase, RPT)], gsem[0])
    pltpu.async_copy(table_hbm.at[pl.ds(rbase, RPT)], table_sh.at[pl.ds(rbase, RPT)], gsem[1])
    pltpu.async_copy(ei_hbm.at[0, pl.ds(wid * CPW, CPW)], src_v, gsem[2])
    pltpu.async_copy(ei_hbm.at[1, pl.ds(wid * CPW, CPW)], dst_v, gsem[3])
    pltpu.make_async_copy(zeros32_hbm.at[pl.ds(rbase, RPT)], acc_sh.at[pl.ds(rbase, RPT)], gsem[0]).wait()
    pltpu.make_async_copy(table_hbm.at[pl.ds(rbase, RPT)], table_sh.at[pl.ds(rbase, RPT)], gsem[1]).wait()
    pltpu.make_async_copy(ei_hbm.at[0, pl.ds(wid * CPW, CPW)], src_v, gsem[2]).wait()
    pltpu.make_async_copy(ei_hbm.at[1, pl.ds(wid * CPW, CPW)], dst_v, gsem[3]).wait()
    plsc.subcore_barrier()
    _edge_ring(table_sh, acc_sh, src_v, dst_v, rows, gsem, ssem, NBP, GDP)
    plsc.subcore_barrier()
    pltpu.sync_copy(acc_sh.at[pl.ds(rbase, RPT)], out_hbm.at[c, pl.ds(rbase, RPT)])


def _elu(a):
    return jnp.where(a > 0, a, jnp.exp(jnp.minimum(a, 0.0)) - 1.0)


def _tc_stage1(degp_hbm, x_ref, w1_ref, table_hbm, dinv_ref,
               degp_v, tbl_v, sem, sem2):
    pltpu.async_copy(degp_hbm, degp_v, sem).wait()
    degs = lax.dot_general(degp_v[...], jnp.ones((NW, 1), jnp.float32),
                           (((0,), (0,)), ((), ())),
                           preferred_element_type=jnp.float32)
    dinv = lax.rsqrt(degs[0:N])
    dinv_ref[...] = dinv
    t1 = jnp.dot(x_ref[...], w1_ref[...], preferred_element_type=jnp.float32)
    tbl_v[...] = t1 * dinv
    pltpu.async_copy(tbl_v, table_hbm.at[pl.ds(0, N)], sem2).wait()


N4 = N // 4
NP4 = N_PAD // 4


def _tc_stage2(pp_hbm, dinvp_ref, b1_ref, table_hbm, pp_v, tbl_v, sem, sem2):
    pltpu.async_copy(pp_hbm, pp_v, sem).wait()
    dinvp = dinvp_ref[...]
    h1 = _elu((pp_v[0, 0:N4] + pp_v[1, 0:N4]) * dinvp + b1_ref[...])
    tbl_v[...] = h1 * dinvp
    pltpu.async_copy(tbl_v, table_hbm.at[pl.ds(0, N4)], sem2).wait()


def _tc_stage3(pp_hbm, dinvp_ref, w2_ref, b2_ref, w4_ref, table_hbm,
               pp_v, tbl_v, sem, sem2):
    pltpu.async_copy(pp_hbm, pp_v, sem).wait()
    dinvp = dinvp_ref[...]
    p = (pp_v[0, 0:N4] + pp_v[1, 0:N4]) * dinvp
    h2 = _elu(jnp.dot(p, w2_ref[...], preferred_element_type=jnp.float32) + b2_ref[...])
    t3 = jnp.dot(h2, w4_ref[...], preferred_element_type=jnp.float32)
    tbl_v[...] = t3 * dinvp
    pltpu.async_copy(tbl_v, table_hbm.at[pl.ds(0, N4)], sem2).wait()


def _tc_stage4(pp_hbm, dinv_ref, b4_ref, out_ref, pp_v, sem):
    pltpu.async_copy(pp_hbm, pp_v, sem).wait()
    dinv = dinv_ref[...]
    logits = (pp_v[0, 0:N] + pp_v[1, 0:N]) * dinv + b4_ref[...]
    col = lax.broadcasted_iota(jnp.int32, logits.shape, 1)
    z = jnp.where(col < 19, logits, -jnp.inf)
    zmax = jnp.max(z, axis=1, keepdims=True)
    e = jnp.exp(z - zmax)
    out_ref[...] = (e / jnp.sum(e, axis=1, keepdims=True))[:, 0:19]


def _sds(shape):
    return jax.ShapeDtypeStruct(shape, jnp.float32)


_HBM_SPEC = pl.BlockSpec(memory_space=pltpu.HBM)
_VMEM_SPEC = pl.BlockSpec(memory_space=pltpu.VMEM)


def kernel(x, edge_index, W1, b1, W2, b2, W4, b4):
    loop = jnp.arange(N_PAD, dtype=jnp.int32)
    dummy = N + jnp.arange(E_PAD - E - N_PAD, dtype=jnp.int32) % (N_PAD - N)
    ei3d = jnp.concatenate(
        [edge_index, jnp.stack([loop, loop]), jnp.stack([dummy, dummy])],
        axis=1).reshape(2, E_PAD // CHUNK, CHUNK)
    zeros32 = jnp.zeros((N_PAD, 32), jnp.float32)
    W4p = jnp.zeros((64, 32), jnp.float32).at[:, :19].set(W4)
    W2_4 = jnp.zeros((128, 256), jnp.float32)
    W4_4 = jnp.zeros((256, 128), jnp.float32)
    for a in range(4):
        W2_4 = W2_4.at[32 * a:32 * a + 32, 64 * a:64 * a + 64].set(W2)
        W4_4 = W4_4.at[64 * a:64 * a + 64, 32 * a:32 * a + 32].set(W4p)
    b1p = jnp.tile(b1, 4).reshape(1, 128)
    b2p = jnp.tile(b2, 4).reshape(1, 256)
    b4r = jnp.zeros((1, 32), jnp.float32).at[0, :19].set(b4)

    degp = _deg_kernel(ei3d)

    table1, dinv = pl.pallas_call(
        _tc_stage1,
        out_shape=[_sds((N_PAD, 32)), _sds((N, 1))],
        in_specs=[_HBM_SPEC, _VMEM_SPEC, _VMEM_SPEC],
        out_specs=[_HBM_SPEC, _VMEM_SPEC],
        scratch_shapes=[pltpu.VMEM((NW, N_PAD), jnp.float32),
                        pltpu.VMEM((N, 32), jnp.float32),
                        pltpu.SemaphoreType.DMA, pltpu.SemaphoreType.DMA],
    )(degp, x, W1)

    pp1 = _prop_kernel(table1, ei3d, zeros32)

    dinvp = jnp.broadcast_to(dinv, (N, 32)).reshape(N4, 128)

    table2p = pl.pallas_call(
        _tc_stage2,
        out_shape=_sds((NP4, 128)),
        in_specs=[_HBM_SPEC, _VMEM_SPEC, _VMEM_SPEC],
        out_specs=_HBM_SPEC,
        scratch_shapes=[pltpu.VMEM((NC, NP4, 128), jnp.float32),
                        pltpu.VMEM((N4, 128), jnp.float32),
                        pltpu.SemaphoreType.DMA, pltpu.SemaphoreType.DMA],
    )(pp1.reshape(NC, NP4, 128), dinvp, b1p)

    pp2 = _prop_kernel(table2p.reshape(N_PAD, 32), ei3d, zeros32)

    table3p = pl.pallas_call(
        _tc_stage3,
        out_shape=_sds((NP4, 128)),
        in_specs=[_HBM_SPEC] + [_VMEM_SPEC] * 4,
        out_specs=_HBM_SPEC,
        scratch_shapes=[pltpu.VMEM((NC, NP4, 128), jnp.float32),
                        pltpu.VMEM((N4, 128), jnp.float32),
                        pltpu.SemaphoreType.DMA, pltpu.SemaphoreType.DMA],
    )(pp2.reshape(NC, NP4, 128), dinvp, W2_4, b2p, W4_4)

    pp3 = _prop_kernel(table3p.reshape(N_PAD, 32), ei3d, zeros32)

    probs = pl.pallas_call(
        _tc_stage4,
        out_shape=_sds((N, 19)),
        in_specs=[_HBM_SPEC, _VMEM_SPEC, _VMEM_SPEC],
        out_specs=_VMEM_SPEC,
        scratch_shapes=[pltpu.VMEM((NC, N_PAD, 32), jnp.float32),
                        pltpu.SemaphoreType.DMA],
    )(pp3, dinv, b4r)

    return probs


# speedup vs baseline: 1.6160x; 1.0787x over previous
import functools

import jax
import jax.numpy as jnp
from jax import lax
from jax.experimental import pallas as pl
from jax.experimental.pallas import tpu as pltpu
from jax.experimental.pallas import tpu_sc as plsc

N = 10000
E = 320000
N_PAD = 10240
E_PAD = 331776
NC = 2
NS = 16
NW = NC * NS
CHUNK = 128
CPW = E_PAD // (NW * CHUNK)
RPT = N_PAD // NS
NBP = 9
GDP = 4

_mesh = plsc.VectorSubcoreMesh(
    core_axis_name="c", subcore_axis_name="s", num_cores=NC, num_subcores=NS)
_sc_params = pltpu.CompilerParams(use_tc_tiling_on_sc=False)
_sc_params_nolayout = pltpu.CompilerParams(use_tc_tiling_on_sc=False,
                                           needs_layout_passes=False)


@functools.partial(
    pl.kernel,
    out_type=jax.ShapeDtypeStruct((NW, N_PAD), jnp.float32),
    mesh=_mesh,
    scratch_types=[
        pltpu.VMEM((CPW, CHUNK), jnp.int32),
        pltpu.VMEM((N_PAD,), jnp.float32),
        pltpu.SemaphoreType.DMA,
    ],
    compiler_params=_sc_params_nolayout,
)
def _deg_kernel(ei_hbm, out_hbm, idx_v, hist_v, sem):
    c = lax.axis_index("c")
    s = lax.axis_index("s")
    wid = c * NS + s
    pltpu.async_copy(ei_hbm.at[1, pl.ds(wid * CPW, CPW)], idx_v, sem)

    zeros_vec = jnp.zeros((16,), jnp.float32)

    def zbody(i, carry):
        hist_v[pl.ds(i * 16, 16)] = zeros_vec
        return carry

    lax.fori_loop(0, N_PAD // 16, zbody, 0)
    pltpu.make_async_copy(ei_hbm.at[1, pl.ds(wid * CPW, CPW)], idx_v, sem).wait()

    ones_vec = jnp.ones((16,), jnp.float32)

    def body(j, carry):
        for k in range(CHUNK // 16):
            v = idx_v[j, pl.ds(16 * k, 16)]
            plsc.addupdate_scatter(hist_v, [v], ones_vec)
        return carry

    lax.fori_loop(0, CPW, body, 0)
    pltpu.sync_copy(hist_v, out_hbm.at[wid])


def _edge_ring(table_sh, acc_sh, src_v, dst_v, rows, gsem, ssem, nb, gd):
    def gather(jt, b):
        pltpu.async_copy(table_sh.at[src_v.at[jt]], rows[b], gsem[b])

    def gwait(j, b):
        pltpu.make_async_copy(table_sh.at[src_v.at[j]], rows[b], gsem[b]).wait()

    def scatter(j, b):
        pltpu.async_copy(rows[b], acc_sh.at[dst_v.at[j]], ssem[b], add=True)

    def swait(j, b):
        pltpu.make_async_copy(rows[b], acc_sh.at[dst_v.at[j]], ssem[b]).wait()

    for b in range(gd):
        gather(b, b)

    for j in range(nb):
        b = j % nb
        gwait(j, b)
        scatter(j, b)
        jt = j + gd
        bt = jt % nb
        if jt >= nb:
            swait(jt - nb, bt)
        gather(jt, bt)

    def outer(g, carry):
        base = g * nb
        for b in range(nb):
            j = base + b
            gwait(j, b)
            scatter(j, b)
            jt = j + gd
            bt = (b + gd) % nb
            swait(jt - nb, bt)
            gather(jt, bt)
        return carry

    lax.fori_loop(1, CPW // nb - 1, outer, 0)

    for b in range(nb):
        j = CPW - nb + b
        gwait(j, b)
        scatter(j, b)
        jt = j + gd
        if jt < CPW:
            bt = jt % nb
            swait(jt - nb, bt)
            gather(jt, bt)

    for b in range(nb):
        swait(CPW - nb + b, b)


def _prop_scratch(nb):
    return [
        pltpu.VMEM((CPW, CHUNK), jnp.int32),
        pltpu.VMEM((CPW, CHUNK), jnp.int32),
        [pltpu.VMEM((CHUNK, 32), jnp.float32)] * nb,
        [pltpu.SemaphoreType.DMA] * nb,
        [pltpu.SemaphoreType.DMA] * nb,
        pltpu.VMEM_SHARED((N_PAD, 32), jnp.float32),
        pltpu.VMEM_SHARED((N_PAD, 32), jnp.float32),
    ]


@functools.partial(
    pl.kernel,
    out_type=jax.ShapeDtypeStruct((NC, N_PAD, 32), jnp.float32),
    mesh=_mesh,
    scratch_types=_prop_scratch(NBP),
    compiler_params=_sc_params,
)
def _prop_kernel(table_hbm, ei_hbm, zeros32_hbm, out_hbm,
                 src_v, dst_v, rows, gsem, ssem, acc_sh, table_sh):
    c = lax.axis_index("c")
    s = lax.axis_index("s")
    wid = c * NS + s
    rbase = s * RPT
    pltpu.async_copy(zeros32_hbm.at[pl.ds(rbase, RPT)], acc_sh.at[pl.ds(rbase, RPT)], gsem[0])
    pltpu.async_copy(table_hbm.at[pl.ds(rbase, RPT)], table_sh.at[pl.ds(rbase, RPT)], gsem[1])
    pltpu.async_copy(ei_hbm.at[0, pl.ds(wid * CPW, CPW)], src_v, gsem[2])
    pltpu.async_copy(ei_hbm.at[1, pl.ds(wid * CPW, CPW)], dst_v, gsem[3])
    pltpu.make_async_copy(zeros32_hbm.at[pl.ds(rbase, RPT)], acc_sh.at[pl.ds(rbase, RPT)], gsem[0]).wait()
    pltpu.make_async_copy(table_hbm.at[pl.ds(rbase, RPT)], table_sh.at[pl.ds(rbase, RPT)], gsem[1]).wait()
    pltpu.make_async_copy(ei_hbm.at[0, pl.ds(wid * CPW, CPW)], src_v, gsem[2]).wait()
    pltpu.make_async_copy(ei_hbm.at[1, pl.ds(wid * CPW, CPW)], dst_v, gsem[3]).wait()
    plsc.subcore_barrier()
    _edge_ring(table_sh, acc_sh, src_v, dst_v, rows, gsem, ssem, NBP, GDP)
    plsc.subcore_barrier()
    pltpu.sync_copy(acc_sh.at[pl.ds(rbase, RPT)], out_hbm.at[c, pl.ds(rbase, RPT)])


def _elu(a):
    return jnp.where(a > 0, a, jnp.exp(jnp.minimum(a, 0.0)) - 1.0)


def _tc_stage1(degp_hbm, x_ref, w1_ref, table_hbm, dinv_ref,
               degp_v, tbl_v, sem, sem2):
    pltpu.async_copy(degp_hbm, degp_v, sem).wait()
    degs = lax.dot_general(degp_v[...], jnp.ones((NW, 1), jnp.float32),
                           (((0,), (0,)), ((), ())),
                           preferred_element_type=jnp.float32)
    dinv = lax.rsqrt(degs[0:N])
    dinv_ref[...] = dinv
    t1 = jnp.dot(x_ref[...], w1_ref[...], preferred_element_type=jnp.float32)
    tbl_v[...] = t1 * dinv
    pltpu.async_copy(tbl_v, table_hbm.at[pl.ds(0, N)], sem2).wait()


N4 = N // 4
NP4 = N_PAD // 4


def _tc_stage2(pp_hbm, dinvp_ref, b1_ref, table_hbm, pp_v, tbl_v, sem, sem2):
    pltpu.async_copy(pp_hbm, pp_v, sem).wait()
    dinvp = dinvp_ref[...]
    h1 = _elu((pp_v[0, 0:N4] + pp_v[1, 0:N4]) * dinvp + b1_ref[...])
    tbl_v[...] = h1 * dinvp
    pltpu.async_copy(tbl_v, table_hbm.at[pl.ds(0, N4)], sem2).wait()


def _tc_stage3(pp_hbm, dinvp_ref, w2_ref, b2_ref, w4_ref, table_hbm,
               pp_v, tbl_v, sem, sem2):
    pltpu.async_copy(pp_hbm, pp_v, sem).wait()
    dinvp = dinvp_ref[...]
    p = (pp_v[0, 0:N4] + pp_v[1, 0:N4]) * dinvp
    h2 = _elu(jnp.dot(p, w2_ref[...], preferred_element_type=jnp.float32) + b2_ref[...])
    t3 = jnp.dot(h2, w4_ref[...], preferred_element_type=jnp.float32)
    tbl_v[...] = t3 * dinvp
    pltpu.async_copy(tbl_v, table_hbm.at[pl.ds(0, N4)], sem2).wait()


def _tc_stage4(pp_hbm, dinvp_ref, b4_ref, m1_ref, m2_ref, out_ref, pp_v, sem):
    pltpu.async_copy(pp_hbm, pp_v, sem).wait()
    dinvp = dinvp_ref[...]
    logits = (pp_v[0, 0:N4] + pp_v[1, 0:N4]) * dinvp + b4_ref[...]
    col = lax.broadcasted_iota(jnp.int32, logits.shape, 1) % 32
    z = jnp.where(col < 19, logits, -jnp.inf)
    zmax = jnp.max(z, axis=1, keepdims=True)
    e = jnp.exp(z - zmax)
    s4 = jnp.dot(e, m1_ref[...], preferred_element_type=jnp.float32)
    rb = jnp.dot(1.0 / s4, m2_ref[...], preferred_element_type=jnp.float32)
    out_ref[...] = e * rb


def _sds(shape):
    return jax.ShapeDtypeStruct(shape, jnp.float32)


_HBM_SPEC = pl.BlockSpec(memory_space=pltpu.HBM)
_VMEM_SPEC = pl.BlockSpec(memory_space=pltpu.VMEM)


def kernel(x, edge_index, W1, b1, W2, b2, W4, b4):
    loop = jnp.arange(N_PAD, dtype=jnp.int32)
    dummy = N + jnp.arange(E_PAD - E - N_PAD, dtype=jnp.int32) % (N_PAD - N)
    ei3d = jnp.concatenate(
        [edge_index, jnp.stack([loop, loop]), jnp.stack([dummy, dummy])],
        axis=1).reshape(2, E_PAD // CHUNK, CHUNK)
    zeros32 = jnp.zeros((N_PAD, 32), jnp.float32)
    W4p = jnp.zeros((64, 32), jnp.float32).at[:, :19].set(W4)
    W2_4 = jnp.zeros((128, 256), jnp.float32)
    W4_4 = jnp.zeros((256, 128), jnp.float32)
    for a in range(4):
        W2_4 = W2_4.at[32 * a:32 * a + 32, 64 * a:64 * a + 64].set(W2)
        W4_4 = W4_4.at[64 * a:64 * a + 64, 32 * a:32 * a + 32].set(W4p)
    b1p = jnp.tile(b1, 4).reshape(1, 128)
    b2p = jnp.tile(b2, 4).reshape(1, 256)
    b4p = jnp.tile(jnp.zeros((32,), jnp.float32).at[:19].set(b4), 4).reshape(1, 128)
    blk = jnp.zeros((128, 4), jnp.float32)
    for a in range(4):
        blk = blk.at[32 * a:32 * a + 32, a].set(1.0)
    M1 = blk
    M2 = blk.T

    degp = _deg_kernel(ei3d)

    table1, dinv = pl.pallas_call(
        _tc_stage1,
        out_shape=[_sds((N_PAD, 32)), _sds((N, 1))],
        in_specs=[_HBM_SPEC, _VMEM_SPEC, _VMEM_SPEC],
        out_specs=[_HBM_SPEC, _VMEM_SPEC],
        scratch_shapes=[pltpu.VMEM((NW, N_PAD), jnp.float32),
                        pltpu.VMEM((N, 32), jnp.float32),
                        pltpu.SemaphoreType.DMA, pltpu.SemaphoreType.DMA],
    )(degp, x, W1)

    pp1 = _prop_kernel(table1, ei3d, zeros32)

    dinvp = jnp.broadcast_to(dinv, (N, 32)).reshape(N4, 128)

    table2p = pl.pallas_call(
        _tc_stage2,
        out_shape=_sds((NP4, 128)),
        in_specs=[_HBM_SPEC, _VMEM_SPEC, _VMEM_SPEC],
        out_specs=_HBM_SPEC,
        scratch_shapes=[pltpu.VMEM((NC, NP4, 128), jnp.float32),
                        pltpu.VMEM((N4, 128), jnp.float32),
                        pltpu.SemaphoreType.DMA, pltpu.SemaphoreType.DMA],
    )(pp1.reshape(NC, NP4, 128), dinvp, b1p)

    pp2 = _prop_kernel(table2p.reshape(N_PAD, 32), ei3d, zeros32)

    table3p = pl.pallas_call(
        _tc_stage3,
        out_shape=_sds((NP4, 128)),
        in_specs=[_HBM_SPEC] + [_VMEM_SPEC] * 4,
        out_specs=_HBM_SPEC,
        scratch_shapes=[pltpu.VMEM((NC, NP4, 128), jnp.float32),
                        pltpu.VMEM((N4, 128), jnp.float32),
                        pltpu.SemaphoreType.DMA, pltpu.SemaphoreType.DMA],
    )(pp2.reshape(NC, NP4, 128), dinvp, W2_4, b2p, W4_4)

    pp3 = _prop_kernel(table3p.reshape(N_PAD, 32), ei3d, zeros32)

    probs_p = pl.pallas_call(
        _tc_stage4,
        out_shape=_sds((N4, 128)),
        in_specs=[_HBM_SPEC] + [_VMEM_SPEC] * 4,
        out_specs=_VMEM_SPEC,
        scratch_shapes=[pltpu.VMEM((NC, NP4, 128), jnp.float32),
                        pltpu.SemaphoreType.DMA],
    )(pp3.reshape(NC, NP4, 128), dinvp, b4p, M1, M2)

    return probs_p.reshape(N, 32)[:, :19]
